# baseline (speedup 1.0000x reference)
import functools

import jax
import jax.numpy as jnp
from jax import lax
from jax.experimental import pallas as pl
from jax.experimental.pallas import tpu as pltpu
from jax.experimental.pallas import tpu_sc as plsc

NC = 2
NS = 16
LANES = 16
NBUF = 4


def _make_emb(B, L, V, D):
  NW = NC * NS
  BPW = B // NW
  DT = D // 8
  mesh = plsc.VectorSubcoreMesh(
      core_axis_name="c", subcore_axis_name="s",
      num_cores=NC, num_subcores=NS)

  scratch = [
      pltpu.VMEM((L, BPW), jnp.int32),
      pltpu.VMEM((L, D), jnp.float32),
      pltpu.VMEM((NBUF, BPW, D), jnp.float32),
  ] + [pltpu.VMEM((DT, 8 * BPW), jnp.float32)] * NBUF \
    + [pltpu.SemaphoreType.DMA] * (2 * NBUF)

  @functools.partial(
      pl.kernel, mesh=mesh,
      out_type=jax.ShapeDtypeStruct((L, DT, NW, 8 * BPW), jnp.float32),
      scratch_types=scratch,
      compiler_params=pltpu.CompilerParams(use_tc_tiling_on_sc=False,
                                           needs_layout_passes=False))
  def emb(ids_hbm, word_hbm, pos_hbm, out_hbm, idx_v, pos_v, rows_v, *rest):
    obuf = rest[:NBUF]
    gsem = rest[NBUF:2 * NBUF]
    wsem = rest[2 * NBUF:]
    wid = lax.axis_index("s") * NC + lax.axis_index("c")
    b0 = wid * BPW

    pltpu.sync_copy(ids_hbm.at[:, pl.ds(b0, BPW)], idx_v)
    pltpu.sync_copy(pos_hbm, pos_v)

    lane = lax.iota(jnp.int32, LANES)
    idt, inb = [], []
    for dg in range(D // LANES):
      dvec = jnp.full((LANES,), dg * LANES, jnp.int32) + lane
      idt.append(dvec // 8)
      inb.append((dvec % 8) * BPW)

    def start_gather(l, b):
      pltpu.async_copy(word_hbm.at[idx_v.at[l]], rows_v.at[b], gsem[b])

    def wait_gather(b):
      pltpu.make_async_copy(
          word_hbm.at[pl.ds(0, BPW)], rows_v.at[b], gsem[b]).wait()

    def start_write(l, b):
      pltpu.async_copy(obuf[b], out_hbm.at[l, :, wid], wsem[b])

    def wait_write(l, b):
      pltpu.make_async_copy(obuf[b], out_hbm.at[l, :, wid], wsem[b]).wait()

    for b in range(NBUF - 1):
      start_gather(b, b)

    def outer(i, _):
      l0 = i * NBUF
      for b in range(NBUF):
        l = l0 + b
        wait_gather(b)
        pv = [pos_v[l, pl.ds(LANES * j, LANES)] for j in range(D // LANES)]

        def radd(r, _):
          rsp = jnp.full((LANES,), r, jnp.int32)
          for j in range(D // LANES):
            x = rows_v[b, r, pl.ds(LANES * j, LANES)]
            plsc.store_scatter(obuf[b], [idt[j], inb[j] + rsp], x + pv[j])
          return 0

        lax.fori_loop(0, BPW, radd, 0)
        start_write(l, b)
        ln = l + NBUF - 1
        bn = (b + NBUF - 1) % NBUF

        @pl.when(ln < L)
        def _():
          @pl.when(ln >= NBUF)
          def _():
            wait_write(ln - NBUF, bn)
          start_gather(ln, bn)
      return 0

    lax.fori_loop(0, L // NBUF, outer, 0)

    for b in range(NBUF):
      wait_write(L - NBUF + b, b)

  return emb


def kernel(input_ids, word_table, pos_table):
  B, L = input_ids.shape
  V, D = word_table.shape
  NW = NC * NS
  BPW = B // NW
  emb = _make_emb(B, L, V, D)
  out4 = emb(input_ids.astype(jnp.int32).T, word_table, pos_table[:L])
  out5 = out4.reshape(L, D // 8, NW, 8, BPW)
  return out5.transpose(2, 4, 0, 1, 3).reshape(B, L, D)


# speedup vs baseline: 1.4641x; 1.4641x over previous
import functools

import jax
import jax.numpy as jnp
from jax import lax
from jax.experimental import pallas as pl
from jax.experimental.pallas import tpu as pltpu
from jax.experimental.pallas import tpu_sc as plsc

NC = 2
NS = 16
LANES = 16
NBUF = 4


def _make_emb(B, L, V, D):
  NW = NC * NS
  BPW = B // NW
  DT = D // 8
  BPWP = BPW + 9
  mesh = plsc.VectorSubcoreMesh(
      core_axis_name="c", subcore_axis_name="s",
      num_cores=NC, num_subcores=NS)

  scratch = [
      pltpu.VMEM((L, BPW), jnp.int32),
      pltpu.VMEM((L, D), jnp.float32),
      pltpu.VMEM((NBUF, BPW, D), jnp.float32),
  ] + [pltpu.VMEM((DT, 8, BPWP), jnp.float32)] * NBUF \
    + [pltpu.SemaphoreType.DMA] * (2 * NBUF)

  @functools.partial(
      pl.kernel, mesh=mesh,
      out_type=jax.ShapeDtypeStruct((L, DT, NW, 8, BPW), jnp.float32),
      scratch_types=scratch,
      compiler_params=pltpu.CompilerParams(use_tc_tiling_on_sc=False,
                                           needs_layout_passes=False))
  def emb(ids_hbm, word_hbm, pos_hbm, out_hbm, idx_v, pos_v, rows_v, *rest):
    obuf = rest[:NBUF]
    gsem = rest[NBUF:2 * NBUF]
    wsem = rest[2 * NBUF:]
    wid = lax.axis_index("s") * NC + lax.axis_index("c")
    b0 = wid * BPW

    pltpu.sync_copy(ids_hbm.at[:, pl.ds(b0, BPW)], idx_v)
    pltpu.sync_copy(pos_hbm, pos_v)

    lane = lax.iota(jnp.int32, LANES)
    idt, idr = [], []
    for dg in range(D // LANES):
      dvec = jnp.full((LANES,), dg * LANES, jnp.int32) + lane
      idt.append(dvec // 8)
      idr.append(dvec % 8)

    def start_gather(l, b):
      pltpu.async_copy(word_hbm.at[idx_v.at[l]], rows_v.at[b], gsem[b])

    def wait_gather(b):
      pltpu.make_async_copy(
          word_hbm.at[pl.ds(0, BPW)], rows_v.at[b], gsem[b]).wait()

    def start_write(l, b):
      pltpu.async_copy(obuf[b].at[:, :, pl.ds(0, BPW)],
                       out_hbm.at[l, :, wid], wsem[b])

    def wait_write(l, b):
      pltpu.make_async_copy(obuf[b].at[:, :, pl.ds(0, BPW)],
                            out_hbm.at[l, :, wid], wsem[b]).wait()

    for b in range(NBUF - 1):
      start_gather(b, b)

    def outer(i, _):
      l0 = i * NBUF
      for b in range(NBUF):
        l = l0 + b
        wait_gather(b)
        pv = [pos_v[l, pl.ds(LANES * j, LANES)] for j in range(D // LANES)]

        def radd(r, _):
          rsp = jnp.full((LANES,), r, jnp.int32)
          for j in range(D // LANES):
            x = rows_v[b, r, pl.ds(LANES * j, LANES)]
            plsc.store_scatter(obuf[b], [idt[j], idr[j], rsp], x + pv[j])
          return 0

        lax.fori_loop(0, BPW, radd, 0)
        start_write(l, b)
        ln = l + NBUF - 1
        bn = (b + NBUF - 1) % NBUF

        @pl.when(ln < L)
        def _():
          @pl.when(ln >= NBUF)
          def _():
            wait_write(ln - NBUF, bn)
          start_gather(ln, bn)
      return 0

    lax.fori_loop(0, L // NBUF, outer, 0)

    for b in range(NBUF):
      wait_write(L - NBUF + b, b)

  return emb


def kernel(input_ids, word_table, pos_table):
  B, L = input_ids.shape
  V, D = word_table.shape
  NW = NC * NS
  BPW = B // NW
  emb = _make_emb(B, L, V, D)
  out5 = emb(input_ids.astype(jnp.int32).T, word_table, pos_table[:L])
  return out5.transpose(2, 4, 0, 1, 3).reshape(B, L, D)


# speedup vs baseline: 2.2890x; 1.5635x over previous
import functools

import jax
import jax.numpy as jnp
from jax import lax
from jax.experimental import pallas as pl
from jax.experimental.pallas import tpu as pltpu
from jax.experimental.pallas import tpu_sc as plsc

NC = 2
NS = 16
LANES = 16
NBUF = 4


def _make_emb(B, L, V, D):
  NW = NC * NS
  BPW = B // NW
  DT = D // 8
  BPWP = BPW + 9
  mesh = plsc.VectorSubcoreMesh(
      core_axis_name="c", subcore_axis_name="s",
      num_cores=NC, num_subcores=NS)

  scratch = [
      pltpu.VMEM((L, BPW), jnp.int32),
      pltpu.VMEM((L, D), jnp.float32),
      pltpu.VMEM((NBUF, BPW, D), jnp.float32),
  ] + [pltpu.VMEM((DT, 8, BPWP), jnp.float32)] * NBUF \
    + [pltpu.SemaphoreType.DMA] * (2 * NBUF)

  @functools.partial(
      pl.kernel, mesh=mesh,
      out_type=jax.ShapeDtypeStruct((L, DT, NW, 8, BPW), jnp.float32),
      scratch_types=scratch,
      compiler_params=pltpu.CompilerParams(use_tc_tiling_on_sc=False,
                                           needs_layout_passes=False))
  def emb(ids_hbm, word_hbm, pos_hbm, out_hbm, idx_v, pos_v, rows_v, *rest):
    obuf = rest[:NBUF]
    gsem = rest[NBUF:2 * NBUF]
    wsem = rest[2 * NBUF:]
    wid = lax.axis_index("s") * NC + lax.axis_index("c")
    b0 = wid * BPW

    pltpu.sync_copy(ids_hbm.at[:, pl.ds(b0, BPW)], idx_v)
    pltpu.sync_copy(pos_hbm, pos_v)

    lane = lax.iota(jnp.int32, LANES)
    idt, idr = [], []
    for dg in range(D // LANES):
      dvec = jnp.full((LANES,), dg * LANES, jnp.int32) + lane
      idt.append(dvec // 8)
      idr.append(dvec % 8)

    def start_gather(l, b):
      pltpu.async_copy(word_hbm.at[idx_v.at[l]], rows_v.at[b], gsem[b])

    def wait_gather(b):
      pltpu.make_async_copy(
          word_hbm.at[pl.ds(0, BPW)], rows_v.at[b], gsem[b]).wait()

    def start_write(l, b):
      pltpu.async_copy(obuf[b].at[:, :, pl.ds(0, BPW)],
                       out_hbm.at[l, :, wid], wsem[b])

    def wait_write(l, b):
      pltpu.make_async_copy(obuf[b].at[:, :, pl.ds(0, BPW)],
                            out_hbm.at[l, :, wid], wsem[b]).wait()

    for b in range(NBUF - 1):
      start_gather(b, b)

    def outer(i, _):
      l0 = i * NBUF
      for b in range(NBUF):
        l = l0 + b
        wait_gather(b)
        pv = [pos_v[l, pl.ds(LANES * j, LANES)] for j in range(D // LANES)]

        @plsc.parallel_loop(0, BPW, unroll=4)
        def radd(r):
          rsp = jnp.full((LANES,), r, jnp.int32)
          for j in range(D // LANES):
            x = rows_v[b, r, pl.ds(LANES * j, LANES)]
            plsc.store_scatter(obuf[b], [idt[j], idr[j], rsp], x + pv[j])
        start_write(l, b)
        ln = l + NBUF - 1
        bn = (b + NBUF - 1) % NBUF

        @pl.when(ln < L)
        def _():
          @pl.when(ln >= NBUF)
          def _():
            wait_write(ln - NBUF, bn)
          start_gather(ln, bn)
      return 0

    lax.fori_loop(0, L // NBUF, outer, 0)

    for b in range(NBUF):
      wait_write(L - NBUF + b, b)

  return emb


def kernel(input_ids, word_table, pos_table):
  B, L = input_ids.shape
  V, D = word_table.shape
  NW = NC * NS
  BPW = B // NW
  emb = _make_emb(B, L, V, D)
  out5 = emb(input_ids.astype(jnp.int32).T, word_table, pos_table[:L])
  return out5.transpose(2, 4, 0, 1, 3).reshape(B, L, D)
